# Initial kernel scaffold; baseline (speedup 1.0000x reference)
import jax
import jax.numpy as jnp
from jax.experimental import pallas as pl


def kernel(features, edges, W1, b1, W2, b2):
    raise NotImplementedError("write your pallas kernel here")


# speedup vs baseline: 4.6038x; 4.6038x over previous
import functools

import jax
import jax.numpy as jnp
from jax import lax
from jax.experimental import pallas as pl
from jax.experimental.pallas import tpu as pltpu
from jax.experimental.pallas import tpu_sc as plsc

N_NODES = 10000
N_EDGES = 320000
D_FEAT = 128
D_HID = 128
D_OUT = 64

NC = 2
NS = 16
NW = NC * NS

EPW = N_EDGES // NW
K = 80
NITER = EPW // K
N_PAD = 10240
RPT = N_PAD // NS
RB = 128


@functools.lru_cache(maxsize=None)
def _make_sc_scatter(D):
    mesh = plsc.VectorSubcoreMesh(core_axis_name="c", subcore_axis_name="s")

    @functools.partial(
        pl.kernel,
        mesh=mesh,
        out_type=jax.ShapeDtypeStruct((NC, N_PAD, D), jnp.float32),
        scratch_types=[
            pltpu.VMEM((K,), jnp.int32),
            pltpu.VMEM((K,), jnp.int32),
            pltpu.VMEM((K, D), jnp.float32),
            pltpu.VMEM((RB, D), jnp.float32),
            pltpu.VMEM_SHARED((N_PAD, D), jnp.float32),
            pltpu.SemaphoreType.DMA,
        ],
    )
    def sc_kernel(y_hbm, src_hbm, dst_hbm, out_hbm,
                  src_v, dst_v, rows_v, buf_v, agg_sh, sem):
        c = lax.axis_index("c")
        s = lax.axis_index("s")
        wid = c * NS + s
        zz = jnp.zeros((16,), jnp.float32)

        @pl.loop(0, RB)
        def _(r):
            @pl.loop(0, D // 16)
            def _(j):
                buf_v[r, pl.ds(j * 16, 16)] = zz

        @pl.loop(0, RPT // RB)
        def _(b):
            pltpu.sync_copy(buf_v, agg_sh.at[pl.ds(s * RPT + b * RB, RB)])

        plsc.subcore_barrier()

        @pl.loop(0, NITER)
        def _(i):
            off = wid * EPW + i * K
            pltpu.sync_copy(src_hbm.at[pl.ds(off, K)], src_v)
            pltpu.sync_copy(dst_hbm.at[pl.ds(off, K)], dst_v)
            pltpu.async_copy(y_hbm.at[src_v], rows_v, sem).wait()
            pltpu.sync_copy(rows_v, agg_sh.at[dst_v], add=True)

        plsc.subcore_barrier()

        @pl.loop(0, RPT // RB)
        def _(b):
            r0 = s * RPT + b * RB
            pltpu.sync_copy(agg_sh.at[pl.ds(r0, RB)], buf_v)
            pltpu.sync_copy(buf_v, out_hbm.at[c, pl.ds(r0, RB)])

    return sc_kernel


_ROW_BLK = 1000
_GRID = N_NODES // _ROW_BLK


def _mm_body(x_ref, w_ref, o_ref):
    o_ref[...] = jnp.dot(x_ref[...], w_ref[...],
                         preferred_element_type=jnp.float32)


def _mid_body(y_ref, p_ref, b_ref, o_ref):
    h = y_ref[...] + p_ref[0] + p_ref[1] + b_ref[...]
    o_ref[...] = jnp.maximum(h, 0.0)


def _final_body(h_ref, q_ref, b_ref, w_ref, o_ref):
    hz = h_ref[...] + q_ref[0] + q_ref[1]
    z = jnp.dot(hz, w_ref[...], preferred_element_type=jnp.float32) + b_ref[...]
    m = jnp.max(z, axis=1, keepdims=True)
    lse = jnp.log(jnp.sum(jnp.exp(z - m), axis=1, keepdims=True)) + m
    o_ref[...] = z - lse


def _mm(x, w):
    n, d = x.shape
    dout = w.shape[1]
    return pl.pallas_call(
        _mm_body,
        grid=(_GRID,),
        in_specs=[
            pl.BlockSpec((_ROW_BLK, d), lambda i: (i, 0)),
            pl.BlockSpec((d, dout), lambda i: (0, 0)),
        ],
        out_specs=pl.BlockSpec((_ROW_BLK, dout), lambda i: (i, 0)),
        out_shape=jax.ShapeDtypeStruct((n, dout), jnp.float32),
    )(x, w)


def _mid(y, p, b):
    n, d = y.shape
    return pl.pallas_call(
        _mid_body,
        grid=(_GRID,),
        in_specs=[
            pl.BlockSpec((_ROW_BLK, d), lambda i: (i, 0)),
            pl.BlockSpec((NC, _ROW_BLK, d), lambda i: (0, i, 0)),
            pl.BlockSpec((1, d), lambda i: (0, 0)),
        ],
        out_specs=pl.BlockSpec((_ROW_BLK, d), lambda i: (i, 0)),
        out_shape=jax.ShapeDtypeStruct((n, d), jnp.float32),
    )(y, p, b)


def _final(h, q, b, w):
    n, d = h.shape
    dout = w.shape[1]
    return pl.pallas_call(
        _final_body,
        grid=(_GRID,),
        in_specs=[
            pl.BlockSpec((_ROW_BLK, d), lambda i: (i, 0)),
            pl.BlockSpec((NC, _ROW_BLK, d), lambda i: (0, i, 0)),
            pl.BlockSpec((1, dout), lambda i: (0, 0)),
            pl.BlockSpec((d, dout), lambda i: (0, 0)),
        ],
        out_specs=pl.BlockSpec((_ROW_BLK, dout), lambda i: (i, 0)),
        out_shape=jax.ShapeDtypeStruct((n, dout), jnp.float32),
    )(h, q, b, w)


def kernel(features, edges, W1, b1, W2, b2):
    src = edges[0].astype(jnp.int32)
    dst = edges[1].astype(jnp.int32)
    b1r = b1.reshape(1, D_HID)
    b2r = b2.reshape(1, D_OUT)

    y1 = _mm(features, W1)
    p = _make_sc_scatter(D_HID)(y1, src, dst)
    h = _mid(y1, p, b1r)
    q = _make_sc_scatter(D_HID)(h, src, dst)
    return _final(h, q, b2r, W2)


# speedup vs baseline: 10.8439x; 2.3554x over previous
import functools

import jax
import jax.numpy as jnp
from jax import lax
from jax.experimental import pallas as pl
from jax.experimental.pallas import tpu as pltpu
from jax.experimental.pallas import tpu_sc as plsc

N_NODES = 10000
N_EDGES = 320000
D_FEAT = 128
D_HID = 128
D_OUT = 64

NC = 2
NS = 16
NW = NC * NS

EPW = N_EDGES // NW
K = 125
NITER = EPW // K
N_PAD = 10240
RPT = N_PAD // NS
RB = 128
DCH = 8


@functools.lru_cache(maxsize=None)
def _make_sc_scatter(D):
    mesh = plsc.VectorSubcoreMesh(core_axis_name="c", subcore_axis_name="s")

    @functools.partial(
        pl.kernel,
        mesh=mesh,
        out_type=jax.ShapeDtypeStruct((NC, N_PAD, D), jnp.float32),
        scratch_types=[
            pltpu.VMEM((NITER, K), jnp.int32),
            pltpu.VMEM((DCH, K), jnp.int32),
            pltpu.VMEM((RB, D), jnp.float32),
            pltpu.VMEM((RB, D), jnp.float32),
            pltpu.VMEM_SHARED((N_PAD, D), jnp.float32),
            pltpu.SemaphoreType.DMA,
            pltpu.SemaphoreType.DMA,
            pltpu.SemaphoreType.DMA,
        ],
    )
    def sc_kernel(y_hbm, src_hbm, dst_hbm, out_hbm,
                  src_v, dst_v, rows_a, rows_b, agg_sh,
                  sem_a, sem_b, sem_i):
        c = lax.axis_index("c")
        s = lax.axis_index("s")
        wid = c * NS + s
        zz = jnp.zeros((16,), jnp.float32)

        idx_cp = pltpu.async_copy(src_hbm.at[wid], src_v, sem_i)

        @pl.loop(0, RB)
        def _(r):
            @pl.loop(0, D // 16)
            def _(j):
                rows_a[r, pl.ds(j * 16, 16)] = zz

        @pl.loop(0, RPT // RB)
        def _(b):
            pltpu.sync_copy(rows_a, agg_sh.at[pl.ds(s * RPT + b * RB, RB)])

        idx_cp.wait()
        plsc.subcore_barrier()

        pltpu.async_copy(y_hbm.at[src_v.at[0]], rows_a.at[pl.ds(0, K)], sem_a)

        @pl.loop(0, NITER // 2)
        def _(j):
            i0 = 2 * j

            @pl.when(lax.rem(j, DCH // 2) == 0)
            def _():
                pltpu.sync_copy(
                    dst_hbm.at[wid, pl.ds(pl.multiple_of(i0, DCH), DCH)], dst_v)

            pltpu.async_copy(y_hbm.at[src_v.at[i0 + 1]],
                             rows_b.at[pl.ds(0, K)], sem_b)
            pltpu.make_async_copy(y_hbm.at[src_v.at[i0]],
                                  rows_a.at[pl.ds(0, K)], sem_a).wait()
            pltpu.sync_copy(rows_a.at[pl.ds(0, K)],
                            agg_sh.at[dst_v.at[lax.rem(i0, DCH)]], add=True)

            @pl.when(j < NITER // 2 - 1)
            def _():
                pltpu.async_copy(y_hbm.at[src_v.at[i0 + 2]],
                                 rows_a.at[pl.ds(0, K)], sem_a)

            pltpu.make_async_copy(y_hbm.at[src_v.at[i0 + 1]],
                                  rows_b.at[pl.ds(0, K)], sem_b).wait()
            pltpu.sync_copy(rows_b.at[pl.ds(0, K)],
                            agg_sh.at[dst_v.at[lax.rem(i0 + 1, DCH)]], add=True)

        plsc.subcore_barrier()

        @pl.loop(0, RPT // RB)
        def _(b):
            r0 = s * RPT + b * RB
            pltpu.sync_copy(agg_sh.at[pl.ds(r0, RB)], rows_a)
            pltpu.sync_copy(rows_a, out_hbm.at[c, pl.ds(r0, RB)])

    return sc_kernel


_ROW_BLK = 1000
_GRID = N_NODES // _ROW_BLK


def _mm_body(x_ref, w_ref, o_ref):
    o_ref[...] = jnp.dot(x_ref[...], w_ref[...],
                         preferred_element_type=jnp.float32)


def _mid_body(y_ref, p_ref, b_ref, o_ref):
    h = y_ref[...] + p_ref[0] + p_ref[1] + b_ref[...]
    o_ref[...] = jnp.maximum(h, 0.0)


def _final_body(h_ref, q_ref, b_ref, w_ref, o_ref):
    hz = h_ref[...] + q_ref[0] + q_ref[1]
    z = jnp.dot(hz, w_ref[...], preferred_element_type=jnp.float32) + b_ref[...]
    m = jnp.max(z, axis=1, keepdims=True)
    lse = jnp.log(jnp.sum(jnp.exp(z - m), axis=1, keepdims=True)) + m
    o_ref[...] = z - lse


def _mm(x, w):
    n, d = x.shape
    dout = w.shape[1]
    return pl.pallas_call(
        _mm_body,
        grid=(_GRID,),
        in_specs=[
            pl.BlockSpec((_ROW_BLK, d), lambda i: (i, 0)),
            pl.BlockSpec((d, dout), lambda i: (0, 0)),
        ],
        out_specs=pl.BlockSpec((_ROW_BLK, dout), lambda i: (i, 0)),
        out_shape=jax.ShapeDtypeStruct((n, dout), jnp.float32),
    )(x, w)


def _mid(y, p, b):
    n, d = y.shape
    return pl.pallas_call(
        _mid_body,
        grid=(_GRID,),
        in_specs=[
            pl.BlockSpec((_ROW_BLK, d), lambda i: (i, 0)),
            pl.BlockSpec((NC, _ROW_BLK, d), lambda i: (0, i, 0)),
            pl.BlockSpec((1, d), lambda i: (0, 0)),
        ],
        out_specs=pl.BlockSpec((_ROW_BLK, d), lambda i: (i, 0)),
        out_shape=jax.ShapeDtypeStruct((n, d), jnp.float32),
    )(y, p, b)


def _final(h, q, b, w):
    n, d = h.shape
    dout = w.shape[1]
    return pl.pallas_call(
        _final_body,
        grid=(_GRID,),
        in_specs=[
            pl.BlockSpec((_ROW_BLK, d), lambda i: (i, 0)),
            pl.BlockSpec((NC, _ROW_BLK, d), lambda i: (0, i, 0)),
            pl.BlockSpec((1, dout), lambda i: (0, 0)),
            pl.BlockSpec((d, dout), lambda i: (0, 0)),
        ],
        out_specs=pl.BlockSpec((_ROW_BLK, dout), lambda i: (i, 0)),
        out_shape=jax.ShapeDtypeStruct((n, dout), jnp.float32),
    )(h, q, b, w)


def kernel(features, edges, W1, b1, W2, b2):
    src = edges[0].astype(jnp.int32).reshape(NW, NITER, K)
    dst = edges[1].astype(jnp.int32).reshape(NW, NITER, K)
    b1r = b1.reshape(1, D_HID)
    b2r = b2.reshape(1, D_OUT)

    y1 = _mm(features, W1)
    p = _make_sc_scatter(D_HID)(y1, src, dst)
    h = _mid(y1, p, b1r)
    q = _make_sc_scatter(D_HID)(h, src, dst)
    return _final(h, q, b2r, W2)


# speedup vs baseline: 11.3528x; 1.0469x over previous
import functools

import jax
import jax.numpy as jnp
from jax import lax
from jax.experimental import pallas as pl
from jax.experimental.pallas import tpu as pltpu
from jax.experimental.pallas import tpu_sc as plsc

N_NODES = 10000
N_EDGES = 320000
D_FEAT = 128
D_HID = 128
D_OUT = 64

NC = 2
NS = 16
NW = NC * NS

EPW = N_EDGES // NW
K = 125
NITER = EPW // K
N_PAD = 10240
RPT = N_PAD // NS
RB = 128
DCH = 8


@functools.lru_cache(maxsize=None)
def _make_sc_scatter(D):
    mesh = plsc.VectorSubcoreMesh(core_axis_name="c", subcore_axis_name="s")

    @functools.partial(
        pl.kernel,
        mesh=mesh,
        out_type=jax.ShapeDtypeStruct((NC, N_PAD, D), jnp.float32),
        scratch_types=[
            pltpu.VMEM((NITER, K), jnp.int32),
            pltpu.VMEM((DCH, K), jnp.int32),
            pltpu.VMEM((RB, D), jnp.float32),
            pltpu.VMEM((RB, D), jnp.float32),
            pltpu.VMEM_SHARED((N_PAD, D), jnp.float32),
            pltpu.SemaphoreType.DMA,
            pltpu.SemaphoreType.DMA,
            pltpu.SemaphoreType.DMA,
        ],
    )
    def sc_kernel(y_hbm, src_hbm, dst_hbm, out_hbm,
                  src_v, dst_v, rows_a, rows_b, agg_sh,
                  sem_a, sem_b, sem_i):
        c = lax.axis_index("c")
        s = lax.axis_index("s")
        wid = c * NS + s
        zz = jnp.zeros((16,), jnp.float32)

        idx_cp = pltpu.async_copy(src_hbm.at[wid], src_v, sem_i)

        @pl.loop(0, RB)
        def _(r):
            @pl.loop(0, D // 16)
            def _(j):
                rows_b[r, pl.ds(j * 16, 16)] = zz

        for b in range(RPT // RB):
            pltpu.async_copy(
                rows_b, agg_sh.at[pl.ds(s * RPT + b * RB, RB)], sem_b)

        idx_cp.wait()
        pltpu.async_copy(y_hbm.at[src_v.at[0]], rows_a.at[pl.ds(0, K)], sem_a)

        for b in range(RPT // RB):
            pltpu.make_async_copy(
                rows_b, agg_sh.at[pl.ds(s * RPT + b * RB, RB)], sem_b).wait()
        plsc.subcore_barrier()


        @pl.loop(0, NITER // 2)
        def _(j):
            i0 = 2 * j

            @pl.when(lax.rem(j, DCH // 2) == 0)
            def _():
                pltpu.sync_copy(
                    dst_hbm.at[wid, pl.ds(pl.multiple_of(i0, DCH), DCH)], dst_v)

            pltpu.async_copy(y_hbm.at[src_v.at[i0 + 1]],
                             rows_b.at[pl.ds(0, K)], sem_b)
            pltpu.make_async_copy(y_hbm.at[src_v.at[i0]],
                                  rows_a.at[pl.ds(0, K)], sem_a).wait()
            pltpu.sync_copy(rows_a.at[pl.ds(0, K)],
                            agg_sh.at[dst_v.at[lax.rem(i0, DCH)]], add=True)

            @pl.when(j < NITER // 2 - 1)
            def _():
                pltpu.async_copy(y_hbm.at[src_v.at[i0 + 2]],
                                 rows_a.at[pl.ds(0, K)], sem_a)

            pltpu.make_async_copy(y_hbm.at[src_v.at[i0 + 1]],
                                  rows_b.at[pl.ds(0, K)], sem_b).wait()
            pltpu.sync_copy(rows_b.at[pl.ds(0, K)],
                            agg_sh.at[dst_v.at[lax.rem(i0 + 1, DCH)]], add=True)

        plsc.subcore_barrier()

        bufs = (rows_a, rows_b)
        sems = (sem_a, sem_b)
        nwo = RPT // RB
        for b in range(nwo):
            buf, sem = bufs[b % 2], sems[b % 2]
            r0 = s * RPT + b * RB
            if b >= 2:
                rp = s * RPT + (b - 2) * RB
                pltpu.make_async_copy(
                    buf, out_hbm.at[c, pl.ds(rp, RB)], sem).wait()
            pltpu.sync_copy(agg_sh.at[pl.ds(r0, RB)], buf)
            pltpu.async_copy(buf, out_hbm.at[c, pl.ds(r0, RB)], sem)
        for b in range(max(nwo - 2, 0), nwo):
            buf, sem = bufs[b % 2], sems[b % 2]
            r0 = s * RPT + b * RB
            pltpu.make_async_copy(buf, out_hbm.at[c, pl.ds(r0, RB)], sem).wait()

    return sc_kernel


_ROW_BLK = 1000
_GRID = N_NODES // _ROW_BLK


def _mid_body(x_ref, p_ref, b_ref, w_ref, o_ref):
    g = x_ref[...] + p_ref[0] + p_ref[1]
    z = jnp.dot(g, w_ref[...], preferred_element_type=jnp.float32) + b_ref[...]
    o_ref[...] = jnp.maximum(z, 0.0)


def _final_body(h_ref, q_ref, b_ref, w_ref, o_ref):
    g = h_ref[...] + q_ref[0] + q_ref[1]
    z = jnp.dot(g, w_ref[...], preferred_element_type=jnp.float32) + b_ref[...]
    m = jnp.max(z, axis=1, keepdims=True)
    lse = jnp.log(jnp.sum(jnp.exp(z - m), axis=1, keepdims=True)) + m
    o_ref[...] = z - lse


def _combine(body, x, p, b, w):
    n, d = x.shape
    dout = w.shape[1]
    return pl.pallas_call(
        body,
        grid=(_GRID,),
        in_specs=[
            pl.BlockSpec((_ROW_BLK, d), lambda i: (i, 0)),
            pl.BlockSpec((NC, _ROW_BLK, d), lambda i: (0, i, 0)),
            pl.BlockSpec((1, dout), lambda i: (0, 0)),
            pl.BlockSpec((d, dout), lambda i: (0, 0)),
        ],
        out_specs=pl.BlockSpec((_ROW_BLK, dout), lambda i: (i, 0)),
        out_shape=jax.ShapeDtypeStruct((n, dout), jnp.float32),
    )(x, p, b, w)


def kernel(features, edges, W1, b1, W2, b2):
    src = edges[0].astype(jnp.int32).reshape(NW, NITER, K)
    dst = edges[1].astype(jnp.int32).reshape(NW, NITER, K)
    b1r = b1.reshape(1, D_HID)
    b2r = b2.reshape(1, D_OUT)

    p = _make_sc_scatter(D_FEAT)(features, src, dst)
    h = _combine(_mid_body, features, p, b1r, W1)
    q = _make_sc_scatter(D_HID)(h, src, dst)
    return _combine(_final_body, h, q, b2r, W2)


# speedup vs baseline: 12.3394x; 1.0869x over previous
import functools

import jax
import jax.numpy as jnp
from jax import lax
from jax.experimental import pallas as pl
from jax.experimental.pallas import tpu as pltpu
from jax.experimental.pallas import tpu_sc as plsc

N_NODES = 10000
N_EDGES = 320000
D_FEAT = 128
D_HID = 128
D_OUT = 64

NC = 2
NS = 16
NW = NC * NS

EPW = N_EDGES // NW
K = 80
NITER = EPW // K
N_PAD = 10240
RPT = N_PAD // NS
RB = 80
DCH = 8
DEPTH = 3


@functools.lru_cache(maxsize=None)
def _make_sc_scatter(D):
    mesh = plsc.VectorSubcoreMesh(core_axis_name="c", subcore_axis_name="s")

    @functools.partial(
        pl.kernel,
        mesh=mesh,
        out_type=jax.ShapeDtypeStruct((NC, N_PAD, D), jnp.float32),
        scratch_types=[
            pltpu.VMEM((NITER, K), jnp.int32),
            pltpu.VMEM((DCH, K), jnp.int32),
            pltpu.VMEM((RB, D), jnp.float32),
            pltpu.VMEM((RB, D), jnp.float32),
            pltpu.VMEM((RB, D), jnp.float32),
            pltpu.VMEM_SHARED((N_PAD, D), jnp.float32),
            pltpu.SemaphoreType.DMA,
            pltpu.SemaphoreType.DMA,
            pltpu.SemaphoreType.DMA,
            pltpu.SemaphoreType.DMA,
        ],
    )
    def sc_kernel(y_hbm, src_hbm, dst_hbm, out_hbm,
                  src_v, dst_v, rows_a, rows_b, rows_c, agg_sh,
                  sem_a, sem_b, sem_c, sem_i):
        c = lax.axis_index("c")
        s = lax.axis_index("s")
        wid = c * NS + s
        zz = jnp.zeros((16,), jnp.float32)
        bufs = (rows_a, rows_b, rows_c)
        sems = (sem_a, sem_b, sem_c)

        idx_cp = pltpu.async_copy(src_hbm.at[wid], src_v, sem_i)

        @pl.loop(0, RB)
        def _(r):
            @pl.loop(0, D // 16)
            def _(j):
                rows_c[r, pl.ds(j * 16, 16)] = zz

        for b in range(RPT // RB):
            pltpu.async_copy(
                rows_c, agg_sh.at[pl.ds(s * RPT + b * RB, RB)], sem_c)

        idx_cp.wait()
        pltpu.async_copy(y_hbm.at[src_v.at[0]], rows_a, sem_a)
        pltpu.async_copy(y_hbm.at[src_v.at[1]], rows_b, sem_b)

        for b in range(RPT // RB):
            pltpu.make_async_copy(
                rows_c, agg_sh.at[pl.ds(s * RPT + b * RB, RB)], sem_c).wait()
        pltpu.async_copy(y_hbm.at[src_v.at[2]], rows_c, sem_c)
        plsc.subcore_barrier()

        NFULL = (NITER - 2) // DEPTH

        @pl.loop(0, NFULL)
        def _(j):
            for t in range(DEPTH):
                i = DEPTH * j + t
                buf, sem = bufs[t], sems[t]

                @pl.when(lax.rem(i, DCH) == 0)
                def _():
                    pltpu.sync_copy(
                        dst_hbm.at[wid, pl.ds(pl.multiple_of(i, DCH), DCH)],
                        dst_v)

                pltpu.make_async_copy(y_hbm.at[src_v.at[i]], buf, sem).wait()
                pltpu.sync_copy(buf, agg_sh.at[dst_v.at[lax.rem(i, DCH)]],
                                add=True)
                if t < DEPTH - 1:
                    pltpu.async_copy(y_hbm.at[src_v.at[i + DEPTH]], buf, sem)
                else:
                    @pl.when(j < NFULL - 1)
                    def _():
                        pltpu.async_copy(y_hbm.at[src_v.at[i + DEPTH]],
                                         buf, sem)

        for t, i in ((0, NITER - 2), (1, NITER - 1)):
            buf, sem = bufs[t], sems[t]
            pltpu.make_async_copy(y_hbm.at[src_v.at[i]], buf, sem).wait()
            pltpu.sync_copy(buf, agg_sh.at[dst_v.at[lax.rem(i, DCH)]],
                            add=True)

        plsc.subcore_barrier()

        bufs = (rows_a, rows_b)
        sems = (sem_a, sem_b)
        nwo = RPT // RB
        for b in range(nwo):
            buf, sem = bufs[b % 2], sems[b % 2]
            r0 = s * RPT + b * RB
            if b >= 2:
                rp = s * RPT + (b - 2) * RB
                pltpu.make_async_copy(
                    buf, out_hbm.at[c, pl.ds(rp, RB)], sem).wait()
            pltpu.sync_copy(agg_sh.at[pl.ds(r0, RB)], buf)
            pltpu.async_copy(buf, out_hbm.at[c, pl.ds(r0, RB)], sem)
        for b in range(max(nwo - 2, 0), nwo):
            buf, sem = bufs[b % 2], sems[b % 2]
            r0 = s * RPT + b * RB
            pltpu.make_async_copy(buf, out_hbm.at[c, pl.ds(r0, RB)], sem).wait()

    return sc_kernel


_ROW_BLK = 1000
_GRID = N_NODES // _ROW_BLK


def _mid_body(x_ref, p_ref, b_ref, w_ref, o_ref):
    g = x_ref[...] + p_ref[0] + p_ref[1]
    z = jnp.dot(g, w_ref[...], preferred_element_type=jnp.float32) + b_ref[...]
    o_ref[...] = jnp.maximum(z, 0.0)


def _final_body(h_ref, q_ref, b_ref, w_ref, o_ref):
    g = h_ref[...] + q_ref[0] + q_ref[1]
    z = jnp.dot(g, w_ref[...], preferred_element_type=jnp.float32) + b_ref[...]
    m = jnp.max(z, axis=1, keepdims=True)
    lse = jnp.log(jnp.sum(jnp.exp(z - m), axis=1, keepdims=True)) + m
    o_ref[...] = z - lse


def _combine(body, x, p, b, w):
    n, d = x.shape
    dout = w.shape[1]
    return pl.pallas_call(
        body,
        grid=(_GRID,),
        in_specs=[
            pl.BlockSpec((_ROW_BLK, d), lambda i: (i, 0)),
            pl.BlockSpec((NC, _ROW_BLK, d), lambda i: (0, i, 0)),
            pl.BlockSpec((1, dout), lambda i: (0, 0)),
            pl.BlockSpec((d, dout), lambda i: (0, 0)),
        ],
        out_specs=pl.BlockSpec((_ROW_BLK, dout), lambda i: (i, 0)),
        out_shape=jax.ShapeDtypeStruct((n, dout), jnp.float32),
    )(x, p, b, w)


def kernel(features, edges, W1, b1, W2, b2):
    src = edges[0].astype(jnp.int32).reshape(NW, NITER, K)
    dst = edges[1].astype(jnp.int32).reshape(NW, NITER, K)
    b1r = b1.reshape(1, D_HID)
    b2r = b2.reshape(1, D_OUT)

    p = _make_sc_scatter(D_FEAT)(features, src, dst)
    h = _combine(_mid_body, features, p, b1r, W1)
    q = _make_sc_scatter(D_HID)(h, src, dst)
    return _combine(_final_body, h, q, b2r, W2)


# speedup vs baseline: 13.3218x; 1.0796x over previous
import functools

import jax
import jax.numpy as jnp
from jax import lax
from jax.experimental import pallas as pl
from jax.experimental.pallas import tpu as pltpu
from jax.experimental.pallas import tpu_sc as plsc

N_NODES = 10000
N_EDGES = 320000
D_FEAT = 128
D_HID = 128
D_OUT = 64

NC = 2
NS = 16
NW = NC * NS

EPW = N_EDGES // NW
K = 80
NITER = EPW // K
N_PAD = 10240
RPT = N_PAD // NS
RB = 80
DCH = 8
DEPTH = 3


@functools.lru_cache(maxsize=None)
def _make_sc_scatter(D):
    mesh = plsc.VectorSubcoreMesh(core_axis_name="c", subcore_axis_name="s")

    @functools.partial(
        pl.kernel,
        mesh=mesh,
        out_type=jax.ShapeDtypeStruct((NC, N_PAD, D), jnp.float32),
        scratch_types=[
            pltpu.VMEM((NITER, K), jnp.int32),
            pltpu.VMEM((2, DCH, K), jnp.int32),
            pltpu.VMEM((RB, D), jnp.float32),
            pltpu.VMEM((RB, D), jnp.float32),
            pltpu.VMEM((RB, D), jnp.float32),
            pltpu.VMEM_SHARED((N_PAD, D), jnp.float32),
            pltpu.SemaphoreType.DMA,
            pltpu.SemaphoreType.DMA,
            pltpu.SemaphoreType.DMA,
            pltpu.SemaphoreType.DMA,
            pltpu.SemaphoreType.DMA,
        ],
    )
    def sc_kernel(y_hbm, src_hbm, dst_hbm, out_hbm,
                  src_v, dst_v, rows_a, rows_b, rows_c, agg_sh,
                  sem_a, sem_b, sem_c, sem_i, sem_d):
        c = lax.axis_index("c")
        s = lax.axis_index("s")
        wid = c * NS + s
        zz = jnp.zeros((16,), jnp.float32)
        bufs = (rows_a, rows_b, rows_c)
        sems = (sem_a, sem_b, sem_c)

        idx_cp = pltpu.async_copy(src_hbm.at[wid], src_v, sem_i)

        @pl.loop(0, RB)
        def _(r):
            @pl.loop(0, D // 16)
            def _(j):
                rows_c[r, pl.ds(j * 16, 16)] = zz

        for b in range(RPT // RB):
            pltpu.async_copy(
                rows_c, agg_sh.at[pl.ds(s * RPT + b * RB, RB)], sem_c)

        pltpu.async_copy(dst_hbm.at[wid, pl.ds(0, DCH)], dst_v.at[0], sem_d)
        idx_cp.wait()
        pltpu.async_copy(y_hbm.at[src_v.at[0]], rows_a, sem_a)
        pltpu.async_copy(y_hbm.at[src_v.at[1]], rows_b, sem_b)

        for b in range(RPT // RB):
            pltpu.make_async_copy(
                rows_c, agg_sh.at[pl.ds(s * RPT + b * RB, RB)], sem_c).wait()
        pltpu.async_copy(y_hbm.at[src_v.at[2]], rows_c, sem_c)
        plsc.subcore_barrier()

        NFULL = (NITER - 2) // DEPTH

        @pl.loop(0, NFULL)
        def _(j):
            for t in range(DEPTH):
                i = DEPTH * j + t
                buf, sem = bufs[t], sems[t]
                par = lax.rem(lax.div(i, DCH), 2)

                @pl.when(lax.rem(i, DCH) == 0)
                def _():
                    i8 = pl.multiple_of(i, DCH)
                    pltpu.make_async_copy(
                        dst_hbm.at[wid, pl.ds(i8, DCH)],
                        dst_v.at[par], sem_d).wait()

                    @pl.when(i + DCH < NITER)
                    def _():
                        pltpu.async_copy(
                            dst_hbm.at[wid, pl.ds(i8 + DCH, DCH)],
                            dst_v.at[1 - par], sem_d)

                pltpu.make_async_copy(y_hbm.at[src_v.at[i]], buf, sem).wait()
                pltpu.sync_copy(
                    buf, agg_sh.at[dst_v.at[par, lax.rem(i, DCH)]], add=True)
                if t < DEPTH - 1:
                    pltpu.async_copy(y_hbm.at[src_v.at[i + DEPTH]], buf, sem)
                else:
                    @pl.when(j < NFULL - 1)
                    def _():
                        pltpu.async_copy(y_hbm.at[src_v.at[i + DEPTH]],
                                         buf, sem)

        for t, i in ((0, NITER - 2), (1, NITER - 1)):
            buf, sem = bufs[t], sems[t]
            par = (i // DCH) % 2
            pltpu.make_async_copy(y_hbm.at[src_v.at[i]], buf, sem).wait()
            pltpu.sync_copy(
                buf, agg_sh.at[dst_v.at[par, lax.rem(i, DCH)]], add=True)

        plsc.subcore_barrier()

        bufs = (rows_a, rows_b)
        sems = (sem_a, sem_b)
        nwo = RPT // RB
        for b in range(nwo):
            buf, sem = bufs[b % 2], sems[b % 2]
            r0 = s * RPT + b * RB
            if b >= 2:
                rp = s * RPT + (b - 2) * RB
                pltpu.make_async_copy(
                    buf, out_hbm.at[c, pl.ds(rp, RB)], sem).wait()
            pltpu.sync_copy(agg_sh.at[pl.ds(r0, RB)], buf)
            pltpu.async_copy(buf, out_hbm.at[c, pl.ds(r0, RB)], sem)
        for b in range(max(nwo - 2, 0), nwo):
            buf, sem = bufs[b % 2], sems[b % 2]
            r0 = s * RPT + b * RB
            pltpu.make_async_copy(buf, out_hbm.at[c, pl.ds(r0, RB)], sem).wait()

    return sc_kernel


_ROW_BLK = 2000
_GRID = N_NODES // _ROW_BLK


def _mid_body(x_ref, p_ref, b_ref, w_ref, o_ref):
    g = x_ref[...] + p_ref[0] + p_ref[1]
    z = jnp.dot(g, w_ref[...], preferred_element_type=jnp.float32) + b_ref[...]
    o_ref[...] = jnp.maximum(z, 0.0)


def _final_body(h_ref, q_ref, b_ref, w_ref, o_ref):
    g = h_ref[...] + q_ref[0] + q_ref[1]
    z = jnp.dot(g, w_ref[...], preferred_element_type=jnp.float32) + b_ref[...]
    m = jnp.max(z, axis=1, keepdims=True)
    lse = jnp.log(jnp.sum(jnp.exp(z - m), axis=1, keepdims=True)) + m
    o_ref[...] = z - lse


def _combine(body, x, p, b, w):
    n, d = x.shape
    dout = w.shape[1]
    return pl.pallas_call(
        body,
        grid=(_GRID,),
        in_specs=[
            pl.BlockSpec((_ROW_BLK, d), lambda i: (i, 0)),
            pl.BlockSpec((NC, _ROW_BLK, d), lambda i: (0, i, 0)),
            pl.BlockSpec((1, dout), lambda i: (0, 0)),
            pl.BlockSpec((d, dout), lambda i: (0, 0)),
        ],
        out_specs=pl.BlockSpec((_ROW_BLK, dout), lambda i: (i, 0)),
        out_shape=jax.ShapeDtypeStruct((n, dout), jnp.float32),
    )(x, p, b, w)


def kernel(features, edges, W1, b1, W2, b2):
    src = edges[0].astype(jnp.int32).reshape(NW, NITER, K)
    dst = edges[1].astype(jnp.int32).reshape(NW, NITER, K)
    b1r = b1.reshape(1, D_HID)
    b2r = b2.reshape(1, D_OUT)

    p = _make_sc_scatter(D_FEAT)(features, src, dst)
    h = _combine(_mid_body, features, p, b1r, W1)
    q = _make_sc_scatter(D_HID)(h, src, dst)
    return _combine(_final_body, h, q, b2r, W2)


# speedup vs baseline: 13.6830x; 1.0271x over previous
import functools

import jax
import jax.numpy as jnp
from jax import lax
from jax.experimental import pallas as pl
from jax.experimental.pallas import tpu as pltpu
from jax.experimental.pallas import tpu_sc as plsc

N_NODES = 10000
N_EDGES = 320000
D_FEAT = 128
D_HID = 128
D_OUT = 64

NC = 2
NS = 16
NW = NC * NS

EPW = N_EDGES // NW
K = 80
NITER = EPW // K
N_PAD = 10240
RPT = N_PAD // NS
RB = 80
DCH = 8
SCH = 16
DEPTH = 4


@functools.lru_cache(maxsize=None)
def _make_sc_scatter(D):
    mesh = plsc.VectorSubcoreMesh(core_axis_name="c", subcore_axis_name="s")

    @functools.partial(
        pl.kernel,
        mesh=mesh,
        out_type=jax.ShapeDtypeStruct((NC, N_PAD, D), jnp.float32),
        scratch_types=[
            pltpu.VMEM((2, SCH, K), jnp.int32),
            pltpu.VMEM((2, DCH, K), jnp.int32),
            pltpu.VMEM((RB, D), jnp.float32),
            pltpu.VMEM((RB, D), jnp.float32),
            pltpu.VMEM((RB, D), jnp.float32),
            pltpu.VMEM((RB, D), jnp.float32),
            pltpu.VMEM_SHARED((N_PAD, D), jnp.float32),
            pltpu.SemaphoreType.DMA,
            pltpu.SemaphoreType.DMA,
            pltpu.SemaphoreType.DMA,
            pltpu.SemaphoreType.DMA,
            pltpu.SemaphoreType.DMA,
            pltpu.SemaphoreType.DMA,
        ],
    )
    def sc_kernel(y_hbm, src_hbm, dst_hbm, out_hbm,
                  src_v, dst_v, rows_a, rows_b, rows_c, rows_d, agg_sh,
                  sem_a, sem_b, sem_c, sem_dd, sem_i, sem_d):
        c = lax.axis_index("c")
        s = lax.axis_index("s")
        wid = c * NS + s
        zz = jnp.zeros((16,), jnp.float32)
        bufs = (rows_a, rows_b, rows_c, rows_d)
        sems = (sem_a, sem_b, sem_c, sem_dd)

        pltpu.sync_copy(src_hbm.at[wid, pl.ds(0, SCH)], src_v.at[0])
        pltpu.async_copy(src_hbm.at[wid, pl.ds(SCH, SCH)], src_v.at[1], sem_i)
        pltpu.async_copy(dst_hbm.at[wid, pl.ds(0, DCH)], dst_v.at[0], sem_d)

        @pl.loop(0, RB)
        def _(r):
            @pl.loop(0, D // 16)
            def _(j):
                rows_d[r, pl.ds(j * 16, 16)] = zz

        for b in range(RPT // RB):
            pltpu.async_copy(
                rows_d, agg_sh.at[pl.ds(s * RPT + b * RB, RB)], sem_dd)

        for t in range(DEPTH - 1):
            pltpu.async_copy(y_hbm.at[src_v.at[0, t]], bufs[t], sems[t])

        for b in range(RPT // RB):
            pltpu.make_async_copy(
                rows_d, agg_sh.at[pl.ds(s * RPT + b * RB, RB)], sem_dd).wait()
        pltpu.async_copy(y_hbm.at[src_v.at[0, DEPTH - 1]],
                         rows_d, sem_dd)
        plsc.subcore_barrier()

        NFULL = (NITER - 1) // DEPTH

        @pl.loop(0, NFULL)
        def _(j):
            for t in range(DEPTH):
                i = DEPTH * j + t
                buf, sem = bufs[t], sems[t]
                par = lax.rem(lax.div(i, DCH), 2)

                @pl.when(lax.rem(i, DCH) == 0)
                def _():
                    i8 = pl.multiple_of(i, DCH)
                    pltpu.make_async_copy(
                        dst_hbm.at[wid, pl.ds(i8, DCH)],
                        dst_v.at[par], sem_d).wait()

                    @pl.when(i + DCH < NITER)
                    def _():
                        pltpu.async_copy(
                            dst_hbm.at[wid, pl.ds(i8 + DCH, DCH)],
                            dst_v.at[1 - par], sem_d)

                g = i + DEPTH
                gpar = lax.rem(lax.div(g, SCH), 2)

                @pl.when((lax.rem(g, SCH) == 0) & (g < NITER))
                def _():
                    pltpu.make_async_copy(
                        src_hbm.at[wid, pl.ds(pl.multiple_of(g, SCH), SCH)],
                        src_v.at[gpar], sem_i).wait()

                @pl.when((lax.rem(g, SCH) == DEPTH) &
                         (g - DEPTH + SCH < NITER))
                def _():
                    g16 = pl.multiple_of(g - DEPTH, SCH)
                    pltpu.async_copy(
                        src_hbm.at[wid, pl.ds(g16 + SCH, SCH)],
                        src_v.at[1 - gpar], sem_i)

                ipar = lax.rem(lax.div(i, SCH), 2)
                pltpu.make_async_copy(
                    y_hbm.at[src_v.at[ipar, lax.rem(i, SCH)]], buf, sem).wait()
                pltpu.sync_copy(
                    buf, agg_sh.at[dst_v.at[par, lax.rem(i, DCH)]], add=True)

                @pl.when(g < NITER)
                def _():
                    pltpu.async_copy(
                        y_hbm.at[src_v.at[gpar, lax.rem(g, SCH)]], buf, sem)

        for t, i in ((0, NITER - 1),):
            buf, sem = bufs[t], sems[t]
            par = (i // DCH) % 2
            ipar = (i // SCH) % 2
            pltpu.make_async_copy(
                y_hbm.at[src_v.at[ipar, lax.rem(i, SCH)]], buf, sem).wait()
            pltpu.sync_copy(
                buf, agg_sh.at[dst_v.at[par, lax.rem(i, DCH)]], add=True)

        plsc.subcore_barrier()

        bufs = (rows_a, rows_b)
        sems = (sem_a, sem_b)
        nwo = RPT // RB
        for b in range(nwo):
            buf, sem = bufs[b % 2], sems[b % 2]
            r0 = s * RPT + b * RB
            if b >= 2:
                rp = s * RPT + (b - 2) * RB
                pltpu.make_async_copy(
                    buf, out_hbm.at[c, pl.ds(rp, RB)], sem).wait()
            pltpu.sync_copy(agg_sh.at[pl.ds(r0, RB)], buf)
            pltpu.async_copy(buf, out_hbm.at[c, pl.ds(r0, RB)], sem)
        for b in range(max(nwo - 2, 0), nwo):
            buf, sem = bufs[b % 2], sems[b % 2]
            r0 = s * RPT + b * RB
            pltpu.make_async_copy(buf, out_hbm.at[c, pl.ds(r0, RB)], sem).wait()

    return sc_kernel


_ROW_BLK = 2000
_GRID = N_NODES // _ROW_BLK


def _mid_body(x_ref, p_ref, b_ref, w_ref, o_ref):
    g = x_ref[...] + p_ref[0] + p_ref[1]
    z = jnp.dot(g, w_ref[...], preferred_element_type=jnp.float32) + b_ref[...]
    o_ref[...] = jnp.maximum(z, 0.0)


def _final_body(h_ref, q_ref, b_ref, w_ref, o_ref):
    g = h_ref[...] + q_ref[0] + q_ref[1]
    z = jnp.dot(g, w_ref[...], preferred_element_type=jnp.float32) + b_ref[...]
    m = jnp.max(z, axis=1, keepdims=True)
    lse = jnp.log(jnp.sum(jnp.exp(z - m), axis=1, keepdims=True)) + m
    o_ref[...] = z - lse


def _combine(body, x, p, b, w):
    n, d = x.shape
    dout = w.shape[1]
    return pl.pallas_call(
        body,
        grid=(_GRID,),
        in_specs=[
            pl.BlockSpec((_ROW_BLK, d), lambda i: (i, 0)),
            pl.BlockSpec((NC, _ROW_BLK, d), lambda i: (0, i, 0)),
            pl.BlockSpec((1, dout), lambda i: (0, 0)),
            pl.BlockSpec((d, dout), lambda i: (0, 0)),
        ],
        out_specs=pl.BlockSpec((_ROW_BLK, dout), lambda i: (i, 0)),
        out_shape=jax.ShapeDtypeStruct((n, dout), jnp.float32),
    )(x, p, b, w)


def kernel(features, edges, W1, b1, W2, b2):
    src = edges[0].astype(jnp.int32).reshape(NW, NITER, K)
    dst = edges[1].astype(jnp.int32).reshape(NW, NITER, K)
    b1r = b1.reshape(1, D_HID)
    b2r = b2.reshape(1, D_OUT)

    p = _make_sc_scatter(D_FEAT)(features, src, dst)
    h = _combine(_mid_body, features, p, b1r, W1)
    q = _make_sc_scatter(D_HID)(h, src, dst)
    return _combine(_final_body, h, q, b2r, W2)


# speedup vs baseline: 13.8575x; 1.0128x over previous
import functools

import jax
import jax.numpy as jnp
from jax import lax
from jax.experimental import pallas as pl
from jax.experimental.pallas import tpu as pltpu
from jax.experimental.pallas import tpu_sc as plsc

N_NODES = 10000
N_EDGES = 320000
D_FEAT = 128
D_HID = 128
D_OUT = 64

NC = 2
NS = 16
NW = NC * NS

EPW = N_EDGES // NW
K = 80
NITER = EPW // K
N_PAD = 10240
RPT = N_PAD // NS
RB = 80
DCH = 8
SCH = 16
DEPTH = 4


@functools.lru_cache(maxsize=None)
def _make_sc_scatter(D):
    mesh = plsc.VectorSubcoreMesh(core_axis_name="c", subcore_axis_name="s")

    @functools.partial(
        pl.kernel,
        mesh=mesh,
        out_type=jax.ShapeDtypeStruct((NC, N_PAD, D), jnp.float32),
        scratch_types=[
            pltpu.VMEM((2, SCH, K), jnp.int32),
            pltpu.VMEM((2, DCH, K), jnp.int32),
            pltpu.VMEM((RB, D), jnp.float32),
            pltpu.VMEM((RB, D), jnp.float32),
            pltpu.VMEM((RB, D), jnp.float32),
            pltpu.VMEM((RB, D), jnp.float32),
            pltpu.VMEM_SHARED((N_PAD, D), jnp.float32),
            pltpu.SemaphoreType.DMA,
            pltpu.SemaphoreType.DMA,
            pltpu.SemaphoreType.DMA,
            pltpu.SemaphoreType.DMA,
            pltpu.SemaphoreType.DMA,
            pltpu.SemaphoreType.DMA,
        ],
    )
    def sc_kernel(y_hbm, src_hbm, dst_hbm, out_hbm,
                  src_v, dst_v, rows_a, rows_b, rows_c, rows_d, agg_sh,
                  sem_a, sem_b, sem_c, sem_dd, sem_i, sem_d):
        c = lax.axis_index("c")
        s = lax.axis_index("s")
        wid = c * NS + s
        zz = jnp.zeros((16,), jnp.float32)
        bufs = (rows_a, rows_b, rows_c, rows_d)
        sems = (sem_a, sem_b, sem_c, sem_dd)

        pltpu.sync_copy(src_hbm.at[wid, pl.ds(0, SCH)], src_v.at[0])
        pltpu.async_copy(src_hbm.at[wid, pl.ds(SCH, SCH)], src_v.at[1], sem_i)
        pltpu.async_copy(dst_hbm.at[wid, pl.ds(0, DCH)], dst_v.at[0], sem_d)

        @pl.loop(0, RB)
        def _(r):
            @pl.loop(0, D // 16)
            def _(j):
                rows_d[r, pl.ds(j * 16, 16)] = zz

        for b in range(RPT // RB):
            pltpu.async_copy(
                rows_d, agg_sh.at[pl.ds(s * RPT + b * RB, RB)], sem_dd)

        for t in range(DEPTH - 1):
            pltpu.async_copy(y_hbm.at[src_v.at[0, t]], bufs[t], sems[t])

        for b in range(RPT // RB):
            pltpu.make_async_copy(
                rows_d, agg_sh.at[pl.ds(s * RPT + b * RB, RB)], sem_dd).wait()
        pltpu.async_copy(y_hbm.at[src_v.at[0, DEPTH - 1]],
                         rows_d, sem_dd)
        plsc.subcore_barrier()

        NFULL = (NITER - 1) // DEPTH

        @pl.loop(0, NFULL)
        def _(j):
            for t in range(DEPTH):
                i = DEPTH * j + t
                buf, sem = bufs[t], sems[t]
                par = lax.rem(lax.div(i, DCH), 2)

                @pl.when(lax.rem(i, DCH) == 0)
                def _():
                    i8 = pl.multiple_of(i, DCH)
                    pltpu.make_async_copy(
                        dst_hbm.at[wid, pl.ds(i8, DCH)],
                        dst_v.at[par], sem_d).wait()

                    @pl.when(i + DCH < NITER)
                    def _():
                        pltpu.async_copy(
                            dst_hbm.at[wid, pl.ds(i8 + DCH, DCH)],
                            dst_v.at[1 - par], sem_d)

                g = i + DEPTH
                gpar = lax.rem(lax.div(g, SCH), 2)

                @pl.when((lax.rem(g, SCH) == 0) & (g < NITER))
                def _():
                    pltpu.make_async_copy(
                        src_hbm.at[wid, pl.ds(pl.multiple_of(g, SCH), SCH)],
                        src_v.at[gpar], sem_i).wait()

                @pl.when((lax.rem(g, SCH) == DEPTH) &
                         (g - DEPTH + SCH < NITER))
                def _():
                    g16 = pl.multiple_of(g - DEPTH, SCH)
                    pltpu.async_copy(
                        src_hbm.at[wid, pl.ds(g16 + SCH, SCH)],
                        src_v.at[1 - gpar], sem_i)

                ipar = lax.rem(lax.div(i, SCH), 2)
                pltpu.make_async_copy(
                    y_hbm.at[src_v.at[ipar, lax.rem(i, SCH)]], buf, sem).wait()
                pltpu.sync_copy(
                    buf, agg_sh.at[dst_v.at[par, lax.rem(i, DCH)]], add=True)

                @pl.when(g < NITER)
                def _():
                    pltpu.async_copy(
                        y_hbm.at[src_v.at[gpar, lax.rem(g, SCH)]], buf, sem)

        for t, i in ((0, NITER - 1),):
            buf, sem = bufs[t], sems[t]
            par = (i // DCH) % 2
            ipar = (i // SCH) % 2
            pltpu.make_async_copy(
                y_hbm.at[src_v.at[ipar, lax.rem(i, SCH)]], buf, sem).wait()
            pltpu.sync_copy(
                buf, agg_sh.at[dst_v.at[par, lax.rem(i, DCH)]], add=True)

        plsc.subcore_barrier()

        bufs = (rows_a, rows_b)
        sems = (sem_a, sem_b)
        nwo = RPT // RB
        for b in range(nwo):
            buf, sem = bufs[b % 2], sems[b % 2]
            r0 = s * RPT + b * RB
            if b >= 2:
                rp = s * RPT + (b - 2) * RB
                pltpu.make_async_copy(
                    buf, out_hbm.at[c, pl.ds(rp, RB)], sem).wait()
            pltpu.sync_copy(agg_sh.at[pl.ds(r0, RB)], buf)
            pltpu.async_copy(buf, out_hbm.at[c, pl.ds(r0, RB)], sem)
        for b in range(max(nwo - 2, 0), nwo):
            buf, sem = bufs[b % 2], sems[b % 2]
            r0 = s * RPT + b * RB
            pltpu.make_async_copy(buf, out_hbm.at[c, pl.ds(r0, RB)], sem).wait()

    return sc_kernel


_ROW_BLK = 5000
_GRID = N_NODES // _ROW_BLK


def _mid_body(x_ref, p_ref, b_ref, w_ref, o_ref):
    g = x_ref[...] + p_ref[0] + p_ref[1]
    z = jnp.dot(g, w_ref[...], preferred_element_type=jnp.float32) + b_ref[...]
    o_ref[...] = jnp.maximum(z, 0.0)


def _final_body(h_ref, q_ref, b_ref, w_ref, o_ref):
    g = h_ref[...] + q_ref[0] + q_ref[1]
    z = jnp.dot(g, w_ref[...], preferred_element_type=jnp.float32) + b_ref[...]
    m = jnp.max(z, axis=1, keepdims=True)
    lse = jnp.log(jnp.sum(jnp.exp(z - m), axis=1, keepdims=True)) + m
    o_ref[...] = z - lse


def _combine(body, x, p, b, w):
    n, d = x.shape
    dout = w.shape[1]
    return pl.pallas_call(
        body,
        grid=(_GRID,),
        in_specs=[
            pl.BlockSpec((_ROW_BLK, d), lambda i: (i, 0)),
            pl.BlockSpec((NC, _ROW_BLK, d), lambda i: (0, i, 0)),
            pl.BlockSpec((1, dout), lambda i: (0, 0)),
            pl.BlockSpec((d, dout), lambda i: (0, 0)),
        ],
        out_specs=pl.BlockSpec((_ROW_BLK, dout), lambda i: (i, 0)),
        out_shape=jax.ShapeDtypeStruct((n, dout), jnp.float32),
    )(x, p, b, w)


def kernel(features, edges, W1, b1, W2, b2):
    src = edges[0].astype(jnp.int32).reshape(NW, NITER, K)
    dst = edges[1].astype(jnp.int32).reshape(NW, NITER, K)
    b1r = b1.reshape(1, D_HID)
    b2r = b2.reshape(1, D_OUT)

    p = _make_sc_scatter(D_FEAT)(features, src, dst)
    h = _combine(_mid_body, features, p, b1r, W1)
    q = _make_sc_scatter(D_HID)(h, src, dst)
    return _combine(_final_body, h, q, b2r, W2)
